# Initial kernel scaffold; baseline (speedup 1.0000x reference)
import jax
import jax.numpy as jnp
from jax.experimental import pallas as pl


def kernel(queries, keys, alpha):
    raise NotImplementedError("write your pallas kernel here")


# speedup vs baseline: 1.6297x; 1.6297x over previous
import functools

import jax
import jax.numpy as jnp
from jax import lax
from jax.experimental import pallas as pl
from jax.experimental.pallas import tpu as pltpu
from jax.experimental.pallas import tpu_sc as plsc

Q = 1024
DIM = 16
NKEYS = 100000
KNN = 16
CHUNK = 2048
NCHUNKS = 49
CHANNELS = 64
PAD = 128
BIG_I = 2147483647


def _search_body(q_ref, k_ref, w_ref, topi_ref, topd_ref):
    step = pl.program_id(0)

    @pl.when(step == 0)
    def _init():
        topd_ref[...] = jnp.full_like(topd_ref, jnp.inf)
        topi_ref[...] = jnp.full_like(topi_ref, BIG_I)

    q = q_ref[...]
    k = k_ref[...]
    qs = jnp.sum(q * q, axis=1, keepdims=True)
    ks = jnp.sum(k * k, axis=1)
    d2 = qs + ks[None, :] - 2.0 * lax.dot_general(
        q, k, (((1,), (1,)), ((), ())), preferred_element_type=jnp.float32
    )
    gidx = step * CHUNK + lax.broadcasted_iota(jnp.int32, (1, CHUNK), 1)
    d2 = jnp.where(gidx < NKEYS, d2, jnp.inf)

    run_d = topd_ref[...]
    run_i = topi_ref[...]
    cd = jnp.concatenate([run_d, d2], axis=1)
    ci = jnp.concatenate([run_i, jnp.broadcast_to(gidx, d2.shape)], axis=1)

    new_d = []
    new_i = []
    for _ in range(KNN):
        m = jnp.min(cd, axis=1, keepdims=True)
        is_min = cd == m
        sel = jnp.min(jnp.where(is_min, ci, BIG_I), axis=1, keepdims=True)
        new_d.append(m)
        new_i.append(sel)
        cd = jnp.where(is_min & (ci == sel), jnp.inf, cd)

    nd = jnp.concatenate(new_d, axis=1)
    ni = jnp.concatenate(new_i, axis=1)
    pad_d = jnp.full((Q, PAD - KNN), jnp.inf, jnp.float32)
    pad_i = jnp.full((Q, PAD - KNN), BIG_I, jnp.int32)
    topd_ref[...] = jnp.concatenate([nd, pad_d], axis=1)
    topi_ref[...] = jnp.concatenate([ni, pad_i], axis=1)

    @pl.when(step == NCHUNKS - 1)
    def _finish():
        wnd = jnp.exp(-0.5 * nd)
        lane_k = lax.broadcasted_iota(jnp.int32, (Q, KNN * 16), 1) // 16
        wide = jnp.zeros((Q, KNN * 16), jnp.float32)
        for kk in range(KNN):
            wide = wide + jnp.where(lane_k == kk, wnd[:, kk : kk + 1], 0.0)
        w_ref[...] = wide


def _run_search(queries, keys_padded):
    w, topi, _ = pl.pallas_call(
        _search_body,
        grid=(NCHUNKS,),
        in_specs=[
            pl.BlockSpec((Q, DIM), lambda i: (0, 0)),
            pl.BlockSpec((CHUNK, DIM), lambda i: (i, 0)),
        ],
        out_specs=[
            pl.BlockSpec((Q, KNN * 16), lambda i: (0, 0)),
            pl.BlockSpec((Q, PAD), lambda i: (0, 0)),
            pl.BlockSpec((Q, PAD), lambda i: (0, 0)),
        ],
        out_shape=[
            jax.ShapeDtypeStruct((Q, KNN * 16), jnp.float32),
            jax.ShapeDtypeStruct((Q, PAD), jnp.int32),
            jax.ShapeDtypeStruct((Q, PAD), jnp.float32),
        ],
    )(queries, keys_padded)
    return w, topi[:, :KNN]


def _make_expand():
    info = plsc.get_sparse_core_info()
    nc, ns = info.num_cores, info.num_subcores
    nw = nc * ns
    qw = Q // nw
    rows_w = qw * KNN
    ngather = rows_w // 128
    mesh = plsc.VectorSubcoreMesh(core_axis_name="c", subcore_axis_name="s")

    @functools.partial(
        pl.kernel,
        mesh=mesh,
        out_type=jax.ShapeDtypeStruct((Q, CHANNELS), jnp.float32),
        compiler_params=pltpu.CompilerParams(use_tc_tiling_on_sc=False),
        scratch_types=[
            pltpu.VMEM((ngather, 128), jnp.int32),
            pltpu.VMEM((rows_w, 16), jnp.float32),
            pltpu.VMEM((rows_w, CHANNELS), jnp.float32),
            pltpu.VMEM((qw, CHANNELS), jnp.float32),
            pltpu.SemaphoreType.DMA,
        ],
    )
    def expand(alpha_hbm, idx_hbm, w_hbm, out_hbm, idx_v, w_v, rows_v, out_v, sem):
        wid = lax.axis_index("s") * nc + lax.axis_index("c")
        pltpu.sync_copy(idx_hbm.at[pl.ds(wid * ngather, ngather)], idx_v)
        pltpu.sync_copy(w_hbm.at[pl.ds(wid * rows_w, rows_w)], w_v)
        copies = [
            pltpu.async_copy(
                alpha_hbm.at[idx_v.at[j]],
                rows_v.at[pl.ds(j * 128, 128)],
                sem,
            )
            for j in range(ngather)
        ]
        for c in copies:
            c.wait()

        def body(qq, carry):
            accs = [jnp.zeros((16,), jnp.float32) for _ in range(CHANNELS // 16)]
            for kk in range(KNN):
                i = qq * KNN + kk
                wsp = w_v[i, pl.ds(0, 16)]
                for c in range(CHANNELS // 16):
                    accs[c] = accs[c] + wsp * rows_v[i, pl.ds(c * 16, 16)]
            for c in range(CHANNELS // 16):
                out_v[qq, pl.ds(c * 16, 16)] = accs[c]
            return carry

        lax.fori_loop(0, qw, body, 0)
        pltpu.sync_copy(out_v, out_hbm.at[pl.ds(wid * qw, qw)])

    return expand


def kernel(queries, keys, alpha):
    keys_padded = jnp.concatenate(
        [keys, jnp.zeros((NCHUNKS * CHUNK - NKEYS, DIM), keys.dtype)], axis=0
    )
    w_wide, topi = _run_search(queries, keys_padded)
    idx_rows = topi.reshape(-1, 128)
    w_rows = w_wide.reshape(-1, 16)
    expand = _make_expand()
    out_qc = expand(alpha, idx_rows, w_rows)
    return out_qc.T


# speedup vs baseline: 1.9980x; 1.2260x over previous
import functools

import jax
import jax.numpy as jnp
from jax import lax
from jax.experimental import pallas as pl
from jax.experimental.pallas import tpu as pltpu
from jax.experimental.pallas import tpu_sc as plsc

Q = 1024
DIM = 16
NKEYS = 100000
KNN = 16
CHUNK = 2048
NCHUNKS = 49
CHANNELS = 64
PAD = 128
BIG_I = 2147483647


def _search_body(q_ref, k_ref, w_ref, topi_ref, topd_ref):
    step = pl.program_id(0)

    q = q_ref[...]
    k = k_ref[...]
    qs = jnp.sum(q * q, axis=1, keepdims=True)
    ks = jnp.sum(k * k, axis=1)
    d2 = qs + ks[None, :] - 2.0 * lax.dot_general(
        q, k, (((1,), (1,)), ((), ())), preferred_element_type=jnp.float32
    )
    gidx = step * CHUNK + lax.broadcasted_iota(jnp.int32, (1, CHUNK), 1)
    d2 = jnp.where(gidx < NKEYS, d2, jnp.inf)
    ci = jnp.broadcast_to(gidx, d2.shape)

    pad_d = jnp.full((Q, PAD - KNN), jnp.inf, jnp.float32)
    pad_i = jnp.full((Q, PAD - KNN), BIG_I, jnp.int32)

    @pl.when(step == 0)
    def _first():
        cd = d2
        cdi = ci
        new_d = []
        new_i = []
        for _ in range(KNN):
            m = jnp.min(cd, axis=1, keepdims=True)
            is_min = cd == m
            sel = jnp.min(jnp.where(is_min, cdi, BIG_I), axis=1, keepdims=True)
            new_d.append(m)
            new_i.append(sel)
            cd = jnp.where(is_min & (cdi == sel), jnp.inf, cd)
        topd_ref[...] = jnp.concatenate(new_d + [pad_d], axis=1)
        topi_ref[...] = jnp.concatenate(new_i + [pad_i], axis=1)

    @pl.when(step > 0)
    def _merge():
        run_d = topd_ref[:, :KNN]
        run_i = topi_ref[:, :KNN]
        thresh = run_d[:, KNN - 1 : KNN]
        surv = d2 < thresh
        masked = jnp.where(surv, d2, jnp.inf)
        n_surv = jnp.max(jnp.sum(surv.astype(jnp.int32), axis=1))
        lane = lax.broadcasted_iota(jnp.int32, (Q, KNN), 1)

        def body(_, carry):
            rd, ri, cd = carry
            m = jnp.min(cd, axis=1, keepdims=True)
            is_min = cd == m
            sel = jnp.min(jnp.where(is_min, ci, BIG_I), axis=1, keepdims=True)
            cd = jnp.where(is_min & (ci == sel), jnp.inf, cd)
            less = (rd < m) | ((rd == m) & (ri < sel))
            pos = jnp.sum(less.astype(jnp.int32), axis=1, keepdims=True)
            shift_d = jnp.concatenate([rd[:, :1], rd[:, : KNN - 1]], axis=1)
            shift_i = jnp.concatenate([ri[:, :1], ri[:, : KNN - 1]], axis=1)
            rd = jnp.where(lane < pos, rd, jnp.where(lane == pos, m, shift_d))
            ri = jnp.where(lane < pos, ri, jnp.where(lane == pos, sel, shift_i))
            return rd, ri, cd

        run_d, run_i, _ = lax.fori_loop(0, n_surv, body, (run_d, run_i, masked))
        topd_ref[...] = jnp.concatenate([run_d, pad_d], axis=1)
        topi_ref[...] = jnp.concatenate([run_i, pad_i], axis=1)

    @pl.when(step == NCHUNKS - 1)
    def _finish():
        wnd = jnp.exp(-0.5 * topd_ref[:, :KNN])
        lane_k = lax.broadcasted_iota(jnp.int32, (Q, KNN * 16), 1) // 16
        wide = jnp.zeros((Q, KNN * 16), jnp.float32)
        for kk in range(KNN):
            wide = wide + jnp.where(lane_k == kk, wnd[:, kk : kk + 1], 0.0)
        w_ref[...] = wide


def _run_search(queries, keys_padded):
    w, topi, _ = pl.pallas_call(
        _search_body,
        grid=(NCHUNKS,),
        in_specs=[
            pl.BlockSpec((Q, DIM), lambda i: (0, 0)),
            pl.BlockSpec((CHUNK, DIM), lambda i: (i, 0)),
        ],
        out_specs=[
            pl.BlockSpec((Q, KNN * 16), lambda i: (0, 0)),
            pl.BlockSpec((Q, PAD), lambda i: (0, 0)),
            pl.BlockSpec((Q, PAD), lambda i: (0, 0)),
        ],
        out_shape=[
            jax.ShapeDtypeStruct((Q, KNN * 16), jnp.float32),
            jax.ShapeDtypeStruct((Q, PAD), jnp.int32),
            jax.ShapeDtypeStruct((Q, PAD), jnp.float32),
        ],
    )(queries, keys_padded)
    return w, topi[:, :KNN]


def _make_expand():
    info = plsc.get_sparse_core_info()
    nc, ns = info.num_cores, info.num_subcores
    nw = nc * ns
    qw = Q // nw
    rows_w = qw * KNN
    ngather = rows_w // 128
    mesh = plsc.VectorSubcoreMesh(core_axis_name="c", subcore_axis_name="s")

    @functools.partial(
        pl.kernel,
        mesh=mesh,
        out_type=jax.ShapeDtypeStruct((Q, CHANNELS), jnp.float32),
        compiler_params=pltpu.CompilerParams(use_tc_tiling_on_sc=False),
        scratch_types=[
            pltpu.VMEM((ngather, 128), jnp.int32),
            pltpu.VMEM((rows_w, 16), jnp.float32),
            pltpu.VMEM((rows_w, CHANNELS), jnp.float32),
            pltpu.VMEM((qw, CHANNELS), jnp.float32),
            pltpu.SemaphoreType.DMA,
        ],
    )
    def expand(alpha_hbm, idx_hbm, w_hbm, out_hbm, idx_v, w_v, rows_v, out_v, sem):
        wid = lax.axis_index("s") * nc + lax.axis_index("c")
        pltpu.sync_copy(idx_hbm.at[pl.ds(wid * ngather, ngather)], idx_v)
        pltpu.sync_copy(w_hbm.at[pl.ds(wid * rows_w, rows_w)], w_v)
        copies = [
            pltpu.async_copy(
                alpha_hbm.at[idx_v.at[j]],
                rows_v.at[pl.ds(j * 128, 128)],
                sem,
            )
            for j in range(ngather)
        ]
        for c in copies:
            c.wait()

        def body(qq, carry):
            accs = [jnp.zeros((16,), jnp.float32) for _ in range(CHANNELS // 16)]
            for kk in range(KNN):
                i = qq * KNN + kk
                wsp = w_v[i, pl.ds(0, 16)]
                for c in range(CHANNELS // 16):
                    accs[c] = accs[c] + wsp * rows_v[i, pl.ds(c * 16, 16)]
            for c in range(CHANNELS // 16):
                out_v[qq, pl.ds(c * 16, 16)] = accs[c]
            return carry

        lax.fori_loop(0, qw, body, 0)
        pltpu.sync_copy(out_v, out_hbm.at[pl.ds(wid * qw, qw)])

    return expand


def kernel(queries, keys, alpha):
    keys_padded = jnp.concatenate(
        [keys, jnp.zeros((NCHUNKS * CHUNK - NKEYS, DIM), keys.dtype)], axis=0
    )
    w_wide, topi = _run_search(queries, keys_padded)
    idx_rows = topi.reshape(-1, 128)
    w_rows = w_wide.reshape(-1, 16)
    expand = _make_expand()
    out_qc = expand(alpha, idx_rows, w_rows)
    return out_qc.T


# speedup vs baseline: 2.9058x; 1.4543x over previous
import functools

import jax
import jax.numpy as jnp
from jax import lax
from jax.experimental import pallas as pl
from jax.experimental.pallas import tpu as pltpu
from jax.experimental.pallas import tpu_sc as plsc

Q = 1024
DIM = 16
NKEYS = 100000
KNN = 16
CHUNK = 2048
NCHUNKS = 49
CHANNELS = 64
PAD = 128
BIG_I = 2147483647


def _search_body(q_ref, k_ref, w_ref, topi_ref, topd_ref, masked_ref):
    step = pl.program_id(0)

    q = q_ref[...]
    k = k_ref[...]
    qs = jnp.sum(q * q, axis=1, keepdims=True)
    ks = jnp.sum(k * k, axis=1)
    d2 = qs + ks[None, :] - 2.0 * lax.dot_general(
        q, k, (((1,), (1,)), ((), ())), preferred_element_type=jnp.float32
    )
    gidx = step * CHUNK + lax.broadcasted_iota(jnp.int32, (1, CHUNK), 1)
    d2 = jnp.where(gidx < NKEYS, d2, jnp.inf)
    ci = jnp.broadcast_to(gidx, d2.shape)

    pad_d = jnp.full((Q, PAD - KNN), jnp.inf, jnp.float32)
    pad_i = jnp.full((Q, PAD - KNN), BIG_I, jnp.int32)

    @pl.when(step == 0)
    def _first():
        cd = d2
        cdi = ci
        new_d = []
        new_i = []
        for _ in range(KNN):
            m = jnp.min(cd, axis=1, keepdims=True)
            is_min = cd == m
            sel = jnp.min(jnp.where(is_min, cdi, BIG_I), axis=1, keepdims=True)
            new_d.append(m)
            new_i.append(sel)
            cd = jnp.where(is_min & (cdi == sel), jnp.inf, cd)
        topd_ref[...] = jnp.concatenate(new_d + [pad_d], axis=1)
        topi_ref[...] = jnp.concatenate(new_i + [pad_i], axis=1)

    @pl.when(step > 0)
    def _merge():
        run_d = topd_ref[:, :KNN]
        run_i = topi_ref[:, :KNN]
        thresh = run_d[:, KNN - 1 : KNN]
        surv = d2 < thresh
        masked_ref[...] = jnp.where(surv, d2, jnp.inf)
        n_surv = jnp.max(jnp.sum(surv.astype(jnp.int32), axis=1))
        lane = lax.broadcasted_iota(jnp.int32, (Q, KNN), 1)

        def body(_, carry):
            rd, ri = carry
            cd = masked_ref[...]
            m = jnp.min(cd, axis=1, keepdims=True)
            is_min = cd == m
            sel = jnp.min(jnp.where(is_min, ci, BIG_I), axis=1, keepdims=True)
            masked_ref[...] = jnp.where(is_min & (ci == sel), jnp.inf, cd)
            less = (rd < m) | ((rd == m) & (ri < sel))
            pos = jnp.sum(less.astype(jnp.int32), axis=1, keepdims=True)
            shift_d = jnp.concatenate([rd[:, :1], rd[:, : KNN - 1]], axis=1)
            shift_i = jnp.concatenate([ri[:, :1], ri[:, : KNN - 1]], axis=1)
            rd = jnp.where(lane < pos, rd, jnp.where(lane == pos, m, shift_d))
            ri = jnp.where(lane < pos, ri, jnp.where(lane == pos, sel, shift_i))
            return rd, ri

        run_d, run_i = lax.fori_loop(0, n_surv, body, (run_d, run_i))
        topd_ref[...] = jnp.concatenate([run_d, pad_d], axis=1)
        topi_ref[...] = jnp.concatenate([run_i, pad_i], axis=1)

    @pl.when(step == NCHUNKS - 1)
    def _finish():
        wnd = jnp.exp(-0.5 * topd_ref[:, :KNN])
        lane_k = lax.broadcasted_iota(jnp.int32, (Q, KNN * 16), 1) // 16
        wide = jnp.zeros((Q, KNN * 16), jnp.float32)
        for kk in range(KNN):
            wide = wide + jnp.where(lane_k == kk, wnd[:, kk : kk + 1], 0.0)
        w_ref[...] = wide


def _run_search(queries, keys_padded):
    w, topi, _ = pl.pallas_call(
        _search_body,
        grid=(NCHUNKS,),
        in_specs=[
            pl.BlockSpec((Q, DIM), lambda i: (0, 0)),
            pl.BlockSpec((CHUNK, DIM), lambda i: (i, 0)),
        ],
        out_specs=[
            pl.BlockSpec((Q, KNN * 16), lambda i: (0, 0)),
            pl.BlockSpec((Q, PAD), lambda i: (0, 0)),
            pl.BlockSpec((Q, PAD), lambda i: (0, 0)),
        ],
        out_shape=[
            jax.ShapeDtypeStruct((Q, KNN * 16), jnp.float32),
            jax.ShapeDtypeStruct((Q, PAD), jnp.int32),
            jax.ShapeDtypeStruct((Q, PAD), jnp.float32),
        ],
        scratch_shapes=[pltpu.VMEM((Q, CHUNK), jnp.float32)],
    )(queries, keys_padded)
    return w, topi[:, :KNN]


def _make_expand():
    info = plsc.get_sparse_core_info()
    nc, ns = info.num_cores, info.num_subcores
    nw = nc * ns
    qw = Q // nw
    rows_w = qw * KNN
    ngather = rows_w // 128
    mesh = plsc.VectorSubcoreMesh(core_axis_name="c", subcore_axis_name="s")

    @functools.partial(
        pl.kernel,
        mesh=mesh,
        out_type=jax.ShapeDtypeStruct((Q, CHANNELS), jnp.float32),
        compiler_params=pltpu.CompilerParams(use_tc_tiling_on_sc=False),
        scratch_types=[
            pltpu.VMEM((ngather, 128), jnp.int32),
            pltpu.VMEM((rows_w, 16), jnp.float32),
            pltpu.VMEM((rows_w, CHANNELS), jnp.float32),
            pltpu.VMEM((qw, CHANNELS), jnp.float32),
            pltpu.SemaphoreType.DMA,
        ],
    )
    def expand(alpha_hbm, idx_hbm, w_hbm, out_hbm, idx_v, w_v, rows_v, out_v, sem):
        wid = lax.axis_index("s") * nc + lax.axis_index("c")
        pltpu.sync_copy(idx_hbm.at[pl.ds(wid * ngather, ngather)], idx_v)
        pltpu.sync_copy(w_hbm.at[pl.ds(wid * rows_w, rows_w)], w_v)
        copies = [
            pltpu.async_copy(
                alpha_hbm.at[idx_v.at[j]],
                rows_v.at[pl.ds(j * 128, 128)],
                sem,
            )
            for j in range(ngather)
        ]
        for c in copies:
            c.wait()

        def body(qq, carry):
            accs = [jnp.zeros((16,), jnp.float32) for _ in range(CHANNELS // 16)]
            for kk in range(KNN):
                i = qq * KNN + kk
                wsp = w_v[i, pl.ds(0, 16)]
                for c in range(CHANNELS // 16):
                    accs[c] = accs[c] + wsp * rows_v[i, pl.ds(c * 16, 16)]
            for c in range(CHANNELS // 16):
                out_v[qq, pl.ds(c * 16, 16)] = accs[c]
            return carry

        lax.fori_loop(0, qw, body, 0)
        pltpu.sync_copy(out_v, out_hbm.at[pl.ds(wid * qw, qw)])

    return expand


def kernel(queries, keys, alpha):
    keys_padded = jnp.concatenate(
        [keys, jnp.zeros((NCHUNKS * CHUNK - NKEYS, DIM), keys.dtype)], axis=0
    )
    w_wide, topi = _run_search(queries, keys_padded)
    idx_rows = topi.reshape(-1, 128)
    w_rows = w_wide.reshape(-1, 16)
    expand = _make_expand()
    out_qc = expand(alpha, idx_rows, w_rows)
    return out_qc.T


# speedup vs baseline: 3.3348x; 1.1477x over previous
import functools

import jax
import jax.numpy as jnp
from jax import lax
from jax.experimental import pallas as pl
from jax.experimental.pallas import tpu as pltpu
from jax.experimental.pallas import tpu_sc as plsc

Q = 1024
DIM = 16
NKEYS = 100000
KNN = 16
CHUNK = 1024
NCHUNKS = 98
CHANNELS = 64
PAD = 128
BIG_I = 2147483647


def _search_body(q_ref, k_ref, w_ref, topi_ref, topd_ref, masked_ref):
    step = pl.program_id(0)

    q = q_ref[...]
    k = k_ref[...]
    qs = jnp.sum(q * q, axis=1, keepdims=True)
    gidx = step * CHUNK + lax.broadcasted_iota(jnp.int32, (1, CHUNK), 1)
    ks = jnp.where(gidx < NKEYS, jnp.sum(k * k, axis=1)[None, :], jnp.inf)
    d2 = qs + ks - 2.0 * lax.dot_general(
        q, k, (((1,), (1,)), ((), ())), preferred_element_type=jnp.float32
    )
    ci = jnp.broadcast_to(gidx, d2.shape)

    pad_d = jnp.full((Q, PAD - KNN), jnp.inf, jnp.float32)
    pad_i = jnp.full((Q, PAD - KNN), BIG_I, jnp.int32)

    @pl.when(step == 0)
    def _first():
        cd = d2
        cdi = ci
        new_d = []
        new_i = []
        for _ in range(KNN):
            m = jnp.min(cd, axis=1, keepdims=True)
            is_min = cd == m
            sel = jnp.min(jnp.where(is_min, cdi, BIG_I), axis=1, keepdims=True)
            new_d.append(m)
            new_i.append(sel)
            cd = jnp.where(is_min & (cdi == sel), jnp.inf, cd)
        topd_ref[...] = jnp.concatenate(new_d + [pad_d], axis=1)
        topi_ref[...] = jnp.concatenate(new_i + [pad_i], axis=1)

    @pl.when(step > 0)
    def _merge():
        run_d = topd_ref[:, :KNN]
        run_i = topi_ref[:, :KNN]
        thresh = run_d[:, KNN - 1 : KNN]
        surv = d2 < thresh
        masked_ref[...] = jnp.where(surv, d2, jnp.inf)
        n_surv = jnp.max(jnp.sum(surv.astype(jnp.int32), axis=1))
        lane = lax.broadcasted_iota(jnp.int32, (Q, KNN), 1)

        def body(_, carry):
            rd, ri = carry
            cd = masked_ref[...]
            m = jnp.min(cd, axis=1, keepdims=True)
            is_min = cd == m
            sel = jnp.min(jnp.where(is_min, ci, BIG_I), axis=1, keepdims=True)
            masked_ref[...] = jnp.where(is_min & (ci == sel), jnp.inf, cd)
            less = (rd < m) | ((rd == m) & (ri < sel))
            pos = jnp.sum(less.astype(jnp.int32), axis=1, keepdims=True)
            shift_d = jnp.concatenate([rd[:, :1], rd[:, : KNN - 1]], axis=1)
            shift_i = jnp.concatenate([ri[:, :1], ri[:, : KNN - 1]], axis=1)
            rd = jnp.where(lane < pos, rd, jnp.where(lane == pos, m, shift_d))
            ri = jnp.where(lane < pos, ri, jnp.where(lane == pos, sel, shift_i))
            return rd, ri

        run_d, run_i = lax.fori_loop(0, n_surv, body, (run_d, run_i))
        topd_ref[...] = jnp.concatenate([run_d, pad_d], axis=1)
        topi_ref[...] = jnp.concatenate([run_i, pad_i], axis=1)

    @pl.when(step == NCHUNKS - 1)
    def _finish():
        wnd = jnp.exp(-0.5 * topd_ref[:, :KNN])
        lane_k = lax.broadcasted_iota(jnp.int32, (Q, KNN * 16), 1) // 16
        wide = jnp.zeros((Q, KNN * 16), jnp.float32)
        for kk in range(KNN):
            wide = wide + jnp.where(lane_k == kk, wnd[:, kk : kk + 1], 0.0)
        w_ref[...] = wide


def _run_search(queries, keys_padded):
    w, topi, _ = pl.pallas_call(
        _search_body,
        grid=(NCHUNKS,),
        in_specs=[
            pl.BlockSpec((Q, DIM), lambda i: (0, 0)),
            pl.BlockSpec((CHUNK, DIM), lambda i: (i, 0)),
        ],
        out_specs=[
            pl.BlockSpec((Q, KNN * 16), lambda i: (0, 0)),
            pl.BlockSpec((Q, PAD), lambda i: (0, 0)),
            pl.BlockSpec((Q, PAD), lambda i: (0, 0)),
        ],
        out_shape=[
            jax.ShapeDtypeStruct((Q, KNN * 16), jnp.float32),
            jax.ShapeDtypeStruct((Q, PAD), jnp.int32),
            jax.ShapeDtypeStruct((Q, PAD), jnp.float32),
        ],
        scratch_shapes=[pltpu.VMEM((Q, CHUNK), jnp.float32)],
    )(queries, keys_padded)
    return w, topi[:, :KNN]


def _make_expand():
    info = plsc.get_sparse_core_info()
    nc, ns = info.num_cores, info.num_subcores
    nw = nc * ns
    qw = Q // nw
    rows_w = qw * KNN
    ngather = rows_w // 128
    mesh = plsc.VectorSubcoreMesh(core_axis_name="c", subcore_axis_name="s")

    @functools.partial(
        pl.kernel,
        mesh=mesh,
        out_type=jax.ShapeDtypeStruct((Q, CHANNELS), jnp.float32),
        compiler_params=pltpu.CompilerParams(use_tc_tiling_on_sc=False),
        scratch_types=[
            pltpu.VMEM((ngather, 128), jnp.int32),
            pltpu.VMEM((rows_w, 16), jnp.float32),
            pltpu.VMEM((rows_w, CHANNELS), jnp.float32),
            pltpu.VMEM((qw, CHANNELS), jnp.float32),
            pltpu.SemaphoreType.DMA,
        ],
    )
    def expand(alpha_hbm, idx_hbm, w_hbm, out_hbm, idx_v, w_v, rows_v, out_v, sem):
        wid = lax.axis_index("s") * nc + lax.axis_index("c")
        pltpu.sync_copy(idx_hbm.at[pl.ds(wid * ngather, ngather)], idx_v)
        pltpu.sync_copy(w_hbm.at[pl.ds(wid * rows_w, rows_w)], w_v)
        copies = [
            pltpu.async_copy(
                alpha_hbm.at[idx_v.at[j]],
                rows_v.at[pl.ds(j * 128, 128)],
                sem,
            )
            for j in range(ngather)
        ]
        for c in copies:
            c.wait()

        def body(qq, carry):
            accs = [jnp.zeros((16,), jnp.float32) for _ in range(CHANNELS // 16)]
            for kk in range(KNN):
                i = qq * KNN + kk
                wsp = w_v[i, pl.ds(0, 16)]
                for c in range(CHANNELS // 16):
                    accs[c] = accs[c] + wsp * rows_v[i, pl.ds(c * 16, 16)]
            for c in range(CHANNELS // 16):
                out_v[qq, pl.ds(c * 16, 16)] = accs[c]
            return carry

        lax.fori_loop(0, qw, body, 0)
        pltpu.sync_copy(out_v, out_hbm.at[pl.ds(wid * qw, qw)])

    return expand


def kernel(queries, keys, alpha):
    keys_padded = jnp.concatenate(
        [keys, jnp.zeros((NCHUNKS * CHUNK - NKEYS, DIM), keys.dtype)], axis=0
    )
    w_wide, topi = _run_search(queries, keys_padded)
    idx_rows = topi.reshape(-1, 128)
    w_rows = w_wide.reshape(-1, 16)
    expand = _make_expand()
    out_qc = expand(alpha, idx_rows, w_rows)
    return out_qc.T
